# Initial kernel scaffold; baseline (speedup 1.0000x reference)
import jax
import jax.numpy as jnp
from jax.experimental import pallas as pl


def kernel(heat, edge_index, edge_weight, W, b, a1, gamma, beta, a2):
    raise NotImplementedError("write your pallas kernel here")


# speedup vs baseline: 4.1793x; 4.1793x over previous
import jax
import jax.numpy as jnp
from jax import lax
from jax.experimental import pallas as pl
from jax.experimental.pallas import tpu as pltpu
from jax.experimental.pallas import tpu_sc as plsc

_N = 10000
_D = 128
_E = 320000
_EPS = 1e-5

_NC = 2
_NS = 16
_L = 16
_NW = _NC * _NS
_EPW = _E // _NW
_C = 80
_NCH = _EPW // _C
_RPT = 624


def _sc_agg_body(x_hbm, src_hbm, dst_hbm, ew_hbm, out_hbm,
                 acc, src_v, dst_v, ew_v, rows_v, zbuf, sem):
    cid = lax.axis_index("c")
    sid = lax.axis_index("s")
    wid = cid * _NS + sid

    def _zrow(i, carry):
        for j in range(_D // _L):
            zbuf[i, pl.ds(j * _L, _L)] = jnp.zeros((_L,), jnp.float32)
        return carry
    lax.fori_loop(0, 16, _zrow, 0)

    row0 = sid * _RPT
    def _zcopy(k, carry):
        pltpu.sync_copy(zbuf, acc.at[pl.ds(row0 + k * 16, 16)])
        return carry
    lax.fori_loop(0, _RPT // 16, _zcopy, 0)

    @pl.when(sid == _NS - 1)
    def _():
        pltpu.sync_copy(zbuf, acc.at[pl.ds(_N - 16, 16)])

    plsc.subcore_barrier()

    ebase = wid * _EPW
    def _chunk(k, carry):
        off = pl.multiple_of(ebase + k * _C, 8)
        pltpu.sync_copy(src_hbm.at[pl.ds(off, _C)], src_v)
        pltpu.sync_copy(dst_hbm.at[pl.ds(off, _C)], dst_v)
        pltpu.sync_copy(ew_hbm.at[pl.ds(off, _C)], ew_v)
        pltpu.async_copy(x_hbm.at[src_v], rows_v, sem).wait()

        def _grp(g, c2):
            wv16 = ew_v[pl.ds(g * _L, _L)]
            for l in range(_L):
                wv = jnp.full((_L,), wv16[l], jnp.float32)
                for j in range(_D // _L):
                    sl = pl.ds(j * _L, _L)
                    rows_v[g * _L + l, sl] = rows_v[g * _L + l, sl] * wv
            return c2
        lax.fori_loop(0, _C // _L, _grp, 0)

        pltpu.sync_copy(rows_v, acc.at[dst_v], add=True)
        return carry
    lax.fori_loop(0, _NCH, _chunk, 0)

    plsc.subcore_barrier()

    pltpu.sync_copy(acc.at[pl.ds(row0, _RPT)],
                    out_hbm.at[cid, pl.ds(row0, _RPT)])

    @pl.when(sid == _NS - 1)
    def _():
        pltpu.sync_copy(acc.at[pl.ds(_N - 16, 16)],
                        out_hbm.at[cid, pl.ds(_N - 16, 16)])


def _sc_aggregate(x, src, dst, ew):
    mesh = plsc.VectorSubcoreMesh(core_axis_name="c", subcore_axis_name="s")
    f = pl.kernel(
        _sc_agg_body,
        mesh=mesh,
        out_type=jax.ShapeDtypeStruct((_NC, _N, _D), jnp.float32),
        scratch_types=[
            pltpu.VMEM_SHARED((_N, _D), jnp.float32),
            pltpu.VMEM((_C,), jnp.int32),
            pltpu.VMEM((_C,), jnp.int32),
            pltpu.VMEM((_C,), jnp.float32),
            pltpu.VMEM((_C, _D), jnp.float32),
            pltpu.VMEM((16, _D), jnp.float32),
            pltpu.SemaphoreType.DMA,
        ],
    )
    return f(x, src, dst, ew)


def _tc_tail_body(p_ref, w_ref, b_ref, a1_ref, g_ref, be_ref, a2_ref, o_ref):
    agg = p_ref[0] + p_ref[1]
    h = lax.dot_general(agg, w_ref[...], (((1,), (0,)), ((), ())),
                        preferred_element_type=jnp.float32,
                        precision=lax.Precision.HIGHEST)
    h = h + b_ref[...]
    a1 = a1_ref[0, 0]
    h = jnp.maximum(h, 0.0) + a1 * jnp.minimum(h, 0.0)
    mean = jnp.mean(h, axis=0, keepdims=True)
    var = jnp.mean((h - mean) ** 2, axis=0, keepdims=True)
    h = (h - mean) / jnp.sqrt(var + _EPS) * g_ref[...] + be_ref[...]
    a2 = a2_ref[0, 0]
    o_ref[...] = jnp.maximum(h, 0.0) + a2 * jnp.minimum(h, 0.0)


def _tc_tail(partials, W, b, a1, gamma, beta, a2):
    return pl.pallas_call(
        _tc_tail_body,
        out_shape=jax.ShapeDtypeStruct((_N, _D), jnp.float32),
        in_specs=[
            pl.BlockSpec(memory_space=pltpu.VMEM),
            pl.BlockSpec(memory_space=pltpu.VMEM),
            pl.BlockSpec(memory_space=pltpu.VMEM),
            pl.BlockSpec(memory_space=pltpu.SMEM),
            pl.BlockSpec(memory_space=pltpu.VMEM),
            pl.BlockSpec(memory_space=pltpu.VMEM),
            pl.BlockSpec(memory_space=pltpu.SMEM),
        ],
        out_specs=pl.BlockSpec(memory_space=pltpu.VMEM),
    )(partials, W, b.reshape(1, _D), a1.reshape(1, 1),
      gamma.reshape(1, _D), beta.reshape(1, _D), a2.reshape(1, 1))


def kernel(heat, edge_index, edge_weight, W, b, a1, gamma, beta, a2):
    src = edge_index[0]
    dst = edge_index[1]
    partials = _sc_aggregate(heat, src, dst, edge_weight)
    return _tc_tail(partials, W, b, a1, gamma, beta, a2)
